# baseline (device time: 159517 ns/iter reference)
import jax
import jax.numpy as jnp
from jax import lax
from jax.experimental import pallas as pl
from jax.experimental.pallas import tpu as pltpu

N_DEV = 4
SEG = 2


def kernel(x, w_mat):
    m, k_local = x.shape
    _, n = w_mat.shape
    m_chunk = m // N_DEV
    nh = n // 2
    seg = m_chunk // SEG

    f32 = jnp.float32

    def body(x_ref, w_ref, out_ref,
             pa_ref, pb_ref, ra1_ref, rb1_ref,
             sa2_ref, sb2_ref, ra2_ref, rb2_ref, xch_ref,
             a1_sems, b1_sems, ra1_sems, rb1_sems,
             a2_sems, b2_sems, ra2_sems, rb2_sems,
             xl_sems, out_sems):
        my = lax.axis_index("i")
        xp = 3 - my
        yp = my + 1 - 2 * (my % 2)
        q = 3 - yp

        barrier_sem = pltpu.get_barrier_semaphore()
        for nbr in [xp, yp]:
            pl.semaphore_signal(
                barrier_sem, inc=1,
                device_id=(nbr,), device_id_type=pl.DeviceIdType.MESH,
            )
        pl.semaphore_wait(barrier_sem, 2)

        def load_seg(c, s, slot):
            cp = pltpu.make_async_copy(
                x_ref.at[pl.ds(c * m_chunk + s * seg, seg), :],
                xch_ref.at[slot],
                xl_sems.at[slot],
            )
            cp.start()
            return cp

        def dot_a(slot):
            return jnp.dot(
                xch_ref[slot, :, :], w_ref[:, 0:nh],
                preferred_element_type=f32,
            )

        def dot_b(slot):
            return jnp.dot(
                xch_ref[slot, :, :], w_ref[:, nh:n],
                preferred_element_type=f32,
            )

        def mk_a1(j, s):
            r0 = s * seg
            return pltpu.make_async_remote_copy(
                src_ref=pa_ref.at[j, pl.ds(r0, seg), :],
                dst_ref=ra1_ref.at[j, pl.ds(r0, seg), :],
                send_sem=a1_sems.at[j, s], recv_sem=ra1_sems.at[j, s],
                device_id=(xp,), device_id_type=pl.DeviceIdType.MESH,
            )

        def mk_b1(j, s):
            r0 = s * seg
            return pltpu.make_async_remote_copy(
                src_ref=pb_ref.at[j, pl.ds(r0, seg), :],
                dst_ref=rb1_ref.at[j, pl.ds(r0, seg), :],
                send_sem=b1_sems.at[j, s], recv_sem=rb1_sems.at[j, s],
                device_id=(yp,), device_id_type=pl.DeviceIdType.MESH,
            )

        def mk_a2(s):
            r0 = s * seg
            return pltpu.make_async_remote_copy(
                src_ref=sa2_ref.at[pl.ds(r0, seg), :],
                dst_ref=ra2_ref.at[pl.ds(r0, seg), :],
                send_sem=a2_sems.at[s], recv_sem=ra2_sems.at[s],
                device_id=(yp,), device_id_type=pl.DeviceIdType.MESH,
            )

        def mk_b2(s):
            r0 = s * seg
            return pltpu.make_async_remote_copy(
                src_ref=sb2_ref.at[pl.ds(r0, seg), :],
                dst_ref=rb2_ref.at[pl.ds(r0, seg), :],
                send_sem=b2_sems.at[s], recv_sem=rb2_sems.at[s],
                device_id=(xp,), device_id_type=pl.DeviceIdType.MESH,
            )

        a1 = [[None] * SEG, [None] * SEG]
        b1 = [[None] * SEG, [None] * SEG]

        for s in range(SEG):
            r0 = s * seg
            load_seg(xp, s, 0).wait()
            pa_ref[0, r0:r0 + seg, :] = dot_a(0)
            a1[0][s] = mk_a1(0, s)
            a1[0][s].start()
            sb2_ref[r0:r0 + seg, :] = dot_b(0)
            load_seg(yp, s, 1).wait()
            pb_ref[0, r0:r0 + seg, :] = dot_b(1)
            b1[0][s] = mk_b1(0, s)
            b1[0][s].start()
            sa2_ref[r0:r0 + seg, :] = dot_a(1)

        for s in range(SEG):
            r0 = s * seg
            load_seg(q, s, s % 2).wait()
            pa_ref[1, r0:r0 + seg, :] = dot_a(s % 2)
            a1[1][s] = mk_a1(1, s)
            a1[1][s].start()
            pb_ref[1, r0:r0 + seg, :] = dot_b(s % 2)
            b1[1][s] = mk_b1(1, s)
            b1[1][s].start()

        for s in range(SEG):
            r0 = s * seg
            load_seg(my, s, s % 2).wait()
            a1[0][s].wait_send()
            pa_ref[0, r0:r0 + seg, :] = dot_a(s % 2)
            b1[0][s].wait_send()
            pb_ref[0, r0:r0 + seg, :] = dot_b(s % 2)

        a2 = [None] * SEG
        b2 = [None] * SEG
        for s in range(SEG):
            r0 = s * seg
            a1[1][s].wait()
            sa2_ref[r0:r0 + seg, :] = (
                sa2_ref[r0:r0 + seg, :] + ra1_ref[1, r0:r0 + seg, :]
            )
            a2[s] = mk_a2(s)
            a2[s].start()
            b1[1][s].wait()
            sb2_ref[r0:r0 + seg, :] = (
                sb2_ref[r0:r0 + seg, :] + rb1_ref[1, r0:r0 + seg, :]
            )
            b2[s] = mk_b2(s)
            b2[s].start()

        out_cps = []
        for s in range(SEG):
            r0 = s * seg
            a1[0][s].wait_recv()
            a2[s].wait()
            sa2_ref[r0:r0 + seg, :] = jnp.maximum(
                pa_ref[0, r0:r0 + seg, :] + ra1_ref[0, r0:r0 + seg, :]
                + ra2_ref[r0:r0 + seg, :], 0.0,
            )
            cp = pltpu.make_async_copy(
                sa2_ref.at[pl.ds(r0, seg), :],
                out_ref.at[pl.ds(r0, seg), pl.ds(0, nh)],
                out_sems.at[0, s],
            )
            cp.start()
            out_cps.append(cp)
            b1[0][s].wait_recv()
            b2[s].wait()
            sb2_ref[r0:r0 + seg, :] = jnp.maximum(
                pb_ref[0, r0:r0 + seg, :] + rb1_ref[0, r0:r0 + seg, :]
                + rb2_ref[r0:r0 + seg, :], 0.0,
            )
            cp = pltpu.make_async_copy(
                sb2_ref.at[pl.ds(r0, seg), :],
                out_ref.at[pl.ds(r0, seg), pl.ds(nh, nh)],
                out_sems.at[1, s],
            )
            cp.start()
            out_cps.append(cp)
        for cp in out_cps:
            cp.wait()

    return pl.pallas_call(
        body,
        out_shape=jax.ShapeDtypeStruct((m_chunk, n), f32),
        in_specs=[
            pl.BlockSpec(memory_space=pltpu.MemorySpace.HBM),
            pl.BlockSpec(memory_space=pltpu.VMEM),
        ],
        out_specs=pl.BlockSpec(memory_space=pltpu.MemorySpace.HBM),
        scratch_shapes=[
            pltpu.VMEM((2, m_chunk, nh), f32),
            pltpu.VMEM((2, m_chunk, nh), f32),
            pltpu.VMEM((2, m_chunk, nh), f32),
            pltpu.VMEM((2, m_chunk, nh), f32),
            pltpu.VMEM((m_chunk, nh), f32),
            pltpu.VMEM((m_chunk, nh), f32),
            pltpu.VMEM((m_chunk, nh), f32),
            pltpu.VMEM((m_chunk, nh), f32),
            pltpu.VMEM((2, seg, k_local), f32),
            pltpu.SemaphoreType.DMA((2, SEG)),
            pltpu.SemaphoreType.DMA((2, SEG)),
            pltpu.SemaphoreType.DMA((2, SEG)),
            pltpu.SemaphoreType.DMA((2, SEG)),
            pltpu.SemaphoreType.DMA((SEG,)),
            pltpu.SemaphoreType.DMA((SEG,)),
            pltpu.SemaphoreType.DMA((SEG,)),
            pltpu.SemaphoreType.DMA((SEG,)),
            pltpu.SemaphoreType.DMA((2,)),
            pltpu.SemaphoreType.DMA((2, SEG)),
        ],
        compiler_params=pltpu.CompilerParams(
            collective_id=0,
            vmem_limit_bytes=128 * 1024 * 1024,
        ),
    )(x, w_mat)


# device time: 155721 ns/iter; 1.0244x vs baseline; 1.0244x over previous
import jax
import jax.numpy as jnp
from jax import lax
from jax.experimental import pallas as pl
from jax.experimental.pallas import tpu as pltpu

N_DEV = 4
SEG = 4


def kernel(x, w_mat):
    m, k_local = x.shape
    _, n = w_mat.shape
    m_chunk = m // N_DEV
    nh = n // 2
    seg = m_chunk // SEG

    f32 = jnp.float32

    def body(x_ref, w_ref, out_ref,
             pa_ref, pb_ref, ra1_ref, rb1_ref,
             sa2_ref, sb2_ref, ra2_ref, rb2_ref, xch_ref,
             a1_sems, b1_sems, ra1_sems, rb1_sems,
             a2_sems, b2_sems, ra2_sems, rb2_sems,
             xl_sems, out_sems):
        my = lax.axis_index("i")
        xp = 3 - my
        yp = my + 1 - 2 * (my % 2)
        q = 3 - yp

        barrier_sem = pltpu.get_barrier_semaphore()
        for nbr in [xp, yp]:
            pl.semaphore_signal(
                barrier_sem, inc=1,
                device_id=(nbr,), device_id_type=pl.DeviceIdType.MESH,
            )
        pl.semaphore_wait(barrier_sem, 2)

        def load_seg(c, s, slot):
            cp = pltpu.make_async_copy(
                x_ref.at[pl.ds(c * m_chunk + s * seg, seg), :],
                xch_ref.at[slot],
                xl_sems.at[slot],
            )
            cp.start()
            return cp

        def dot_a(slot):
            return jnp.dot(
                xch_ref[slot, :, :], w_ref[:, 0:nh],
                preferred_element_type=f32,
            )

        def dot_b(slot):
            return jnp.dot(
                xch_ref[slot, :, :], w_ref[:, nh:n],
                preferred_element_type=f32,
            )

        def mk_a1(j, s):
            r0 = s * seg
            return pltpu.make_async_remote_copy(
                src_ref=pa_ref.at[j, pl.ds(r0, seg), :],
                dst_ref=ra1_ref.at[j, pl.ds(r0, seg), :],
                send_sem=a1_sems.at[j, s], recv_sem=ra1_sems.at[j, s],
                device_id=(xp,), device_id_type=pl.DeviceIdType.MESH,
            )

        def mk_b1(j, s):
            r0 = s * seg
            return pltpu.make_async_remote_copy(
                src_ref=pb_ref.at[j, pl.ds(r0, seg), :],
                dst_ref=rb1_ref.at[j, pl.ds(r0, seg), :],
                send_sem=b1_sems.at[j, s], recv_sem=rb1_sems.at[j, s],
                device_id=(yp,), device_id_type=pl.DeviceIdType.MESH,
            )

        def mk_a2(s):
            r0 = s * seg
            return pltpu.make_async_remote_copy(
                src_ref=sa2_ref.at[pl.ds(r0, seg), :],
                dst_ref=ra2_ref.at[pl.ds(r0, seg), :],
                send_sem=a2_sems.at[s], recv_sem=ra2_sems.at[s],
                device_id=(yp,), device_id_type=pl.DeviceIdType.MESH,
            )

        def mk_b2(s):
            r0 = s * seg
            return pltpu.make_async_remote_copy(
                src_ref=sb2_ref.at[pl.ds(r0, seg), :],
                dst_ref=rb2_ref.at[pl.ds(r0, seg), :],
                send_sem=b2_sems.at[s], recv_sem=rb2_sems.at[s],
                device_id=(xp,), device_id_type=pl.DeviceIdType.MESH,
            )

        a1 = [[None] * SEG, [None] * SEG]
        b1 = [[None] * SEG, [None] * SEG]

        for s in range(SEG):
            r0 = s * seg
            load_seg(q, s, s % 2).wait()
            pa_ref[1, r0:r0 + seg, :] = dot_a(s % 2)
            a1[1][s] = mk_a1(1, s)
            a1[1][s].start()
            pb_ref[1, r0:r0 + seg, :] = dot_b(s % 2)
            b1[1][s] = mk_b1(1, s)
            b1[1][s].start()

        for s in range(SEG):
            r0 = s * seg
            load_seg(xp, s, s % 2).wait()
            pa_ref[0, r0:r0 + seg, :] = dot_a(s % 2)
            a1[0][s] = mk_a1(0, s)
            a1[0][s].start()
            sb2_ref[r0:r0 + seg, :] = dot_b(s % 2)
            load_seg(yp, s, 2 + s % 2).wait()
            pb_ref[0, r0:r0 + seg, :] = dot_b(2 + s % 2)
            b1[0][s] = mk_b1(0, s)
            b1[0][s].start()
            sa2_ref[r0:r0 + seg, :] = dot_a(2 + s % 2)

        for s in range(SEG):
            r0 = s * seg
            load_seg(my, s, s % 2).wait()
            a1[0][s].wait_send()
            pa_ref[0, r0:r0 + seg, :] = dot_a(s % 2)
            b1[0][s].wait_send()
            pb_ref[0, r0:r0 + seg, :] = dot_b(s % 2)

        a2 = [None] * SEG
        b2 = [None] * SEG
        for s in range(SEG):
            r0 = s * seg
            a1[1][s].wait()
            sa2_ref[r0:r0 + seg, :] = (
                sa2_ref[r0:r0 + seg, :] + ra1_ref[1, r0:r0 + seg, :]
            )
            a2[s] = mk_a2(s)
            a2[s].start()
            b1[1][s].wait()
            sb2_ref[r0:r0 + seg, :] = (
                sb2_ref[r0:r0 + seg, :] + rb1_ref[1, r0:r0 + seg, :]
            )
            b2[s] = mk_b2(s)
            b2[s].start()

        out_cps = []
        for s in range(SEG):
            r0 = s * seg
            a1[0][s].wait_recv()
            a2[s].wait()
            sa2_ref[r0:r0 + seg, :] = jnp.maximum(
                pa_ref[0, r0:r0 + seg, :] + ra1_ref[0, r0:r0 + seg, :]
                + ra2_ref[r0:r0 + seg, :], 0.0,
            )
            cp = pltpu.make_async_copy(
                sa2_ref.at[pl.ds(r0, seg), :],
                out_ref.at[pl.ds(r0, seg), pl.ds(0, nh)],
                out_sems.at[0, s],
            )
            cp.start()
            out_cps.append(cp)
            b1[0][s].wait_recv()
            b2[s].wait()
            sb2_ref[r0:r0 + seg, :] = jnp.maximum(
                pb_ref[0, r0:r0 + seg, :] + rb1_ref[0, r0:r0 + seg, :]
                + rb2_ref[r0:r0 + seg, :], 0.0,
            )
            cp = pltpu.make_async_copy(
                sb2_ref.at[pl.ds(r0, seg), :],
                out_ref.at[pl.ds(r0, seg), pl.ds(nh, nh)],
                out_sems.at[1, s],
            )
            cp.start()
            out_cps.append(cp)
        for cp in out_cps:
            cp.wait()

    return pl.pallas_call(
        body,
        out_shape=jax.ShapeDtypeStruct((m_chunk, n), f32),
        in_specs=[
            pl.BlockSpec(memory_space=pltpu.MemorySpace.HBM),
            pl.BlockSpec(memory_space=pltpu.VMEM),
        ],
        out_specs=pl.BlockSpec(memory_space=pltpu.MemorySpace.HBM),
        scratch_shapes=[
            pltpu.VMEM((2, m_chunk, nh), f32),
            pltpu.VMEM((2, m_chunk, nh), f32),
            pltpu.VMEM((2, m_chunk, nh), f32),
            pltpu.VMEM((2, m_chunk, nh), f32),
            pltpu.VMEM((m_chunk, nh), f32),
            pltpu.VMEM((m_chunk, nh), f32),
            pltpu.VMEM((m_chunk, nh), f32),
            pltpu.VMEM((m_chunk, nh), f32),
            pltpu.VMEM((4, seg, k_local), f32),
            pltpu.SemaphoreType.DMA((2, SEG)),
            pltpu.SemaphoreType.DMA((2, SEG)),
            pltpu.SemaphoreType.DMA((2, SEG)),
            pltpu.SemaphoreType.DMA((2, SEG)),
            pltpu.SemaphoreType.DMA((SEG,)),
            pltpu.SemaphoreType.DMA((SEG,)),
            pltpu.SemaphoreType.DMA((SEG,)),
            pltpu.SemaphoreType.DMA((SEG,)),
            pltpu.SemaphoreType.DMA((4,)),
            pltpu.SemaphoreType.DMA((2, SEG)),
        ],
        compiler_params=pltpu.CompilerParams(
            collective_id=0,
            vmem_limit_bytes=128 * 1024 * 1024,
        ),
    )(x, w_mat)


# device time: 154522 ns/iter; 1.0323x vs baseline; 1.0078x over previous
import jax
import jax.numpy as jnp
from jax import lax
from jax.experimental import pallas as pl
from jax.experimental.pallas import tpu as pltpu

N_DEV = 4
SEG = 4


def kernel(x, w_mat):
    m, k_local = x.shape
    _, n = w_mat.shape
    m_chunk = m // N_DEV
    nh = n // 2
    seg = m_chunk // SEG

    f32 = jnp.float32

    def body(x_ref, w_ref, out_ref,
             pa_ref, pb_ref, ra1_ref, rb1_ref,
             sa2_ref, sb2_ref, ra2_ref, rb2_ref, xch_ref,
             a1_sems, b1_sems, ra1_sems, rb1_sems,
             a2_sems, b2_sems, ra2_sems, rb2_sems,
             xl_sems, out_sems):
        my = lax.axis_index("i")
        xp = 3 - my
        yp = my + 1 - 2 * (my % 2)
        q = 3 - yp

        barrier_sem = pltpu.get_barrier_semaphore()
        for nbr in [xp, yp]:
            pl.semaphore_signal(
                barrier_sem, inc=1,
                device_id=(nbr,), device_id_type=pl.DeviceIdType.MESH,
            )
        pl.semaphore_wait(barrier_sem, 2)

        def load_seg(c, s, slot):
            cp = pltpu.make_async_copy(
                x_ref.at[pl.ds(c * m_chunk + s * seg, seg), :],
                xch_ref.at[slot],
                xl_sems.at[slot],
            )
            cp.start()
            return cp

        def dot_a(slot):
            return jnp.dot(
                xch_ref[slot, :, :], w_ref[:, 0:nh],
                preferred_element_type=f32,
            )

        def dot_b(slot):
            return jnp.dot(
                xch_ref[slot, :, :], w_ref[:, nh:n],
                preferred_element_type=f32,
            )

        def mk_a1(j, s):
            r0 = s * seg
            return pltpu.make_async_remote_copy(
                src_ref=pa_ref.at[j, pl.ds(r0, seg), :],
                dst_ref=ra1_ref.at[j, pl.ds(r0, seg), :],
                send_sem=a1_sems.at[j, s], recv_sem=ra1_sems.at[j, s],
                device_id=(xp,), device_id_type=pl.DeviceIdType.MESH,
            )

        def mk_b1(j, s):
            r0 = s * seg
            return pltpu.make_async_remote_copy(
                src_ref=pb_ref.at[j, pl.ds(r0, seg), :],
                dst_ref=rb1_ref.at[j, pl.ds(r0, seg), :],
                send_sem=b1_sems.at[j, s], recv_sem=rb1_sems.at[j, s],
                device_id=(yp,), device_id_type=pl.DeviceIdType.MESH,
            )

        def mk_a2(s):
            r0 = s * seg
            return pltpu.make_async_remote_copy(
                src_ref=sa2_ref.at[pl.ds(r0, seg), :],
                dst_ref=ra2_ref.at[pl.ds(r0, seg), :],
                send_sem=a2_sems.at[s], recv_sem=ra2_sems.at[s],
                device_id=(yp,), device_id_type=pl.DeviceIdType.MESH,
            )

        def mk_b2(s):
            r0 = s * seg
            return pltpu.make_async_remote_copy(
                src_ref=sb2_ref.at[pl.ds(r0, seg), :],
                dst_ref=rb2_ref.at[pl.ds(r0, seg), :],
                send_sem=b2_sems.at[s], recv_sem=rb2_sems.at[s],
                device_id=(xp,), device_id_type=pl.DeviceIdType.MESH,
            )

        a1 = [[None] * SEG, [None] * SEG]
        b1 = [[None] * SEG, [None] * SEG]

        for s in range(SEG):
            r0 = s * seg
            load_seg(q, s, s % 2).wait()
            pa_ref[1, r0:r0 + seg, :] = dot_a(s % 2)
            a1[1][s] = mk_a1(1, s)
            a1[1][s].start()
            pb_ref[1, r0:r0 + seg, :] = dot_b(s % 2)
            b1[1][s] = mk_b1(1, s)
            b1[1][s].start()

        for s in range(SEG):
            r0 = s * seg
            load_seg(xp, s, s % 2).wait()
            pa_ref[0, r0:r0 + seg, :] = dot_a(s % 2)
            a1[0][s] = mk_a1(0, s)
            a1[0][s].start()
            sb2_ref[r0:r0 + seg, :] = dot_b(s % 2)
            load_seg(yp, s, 2 + s % 2).wait()
            pb_ref[0, r0:r0 + seg, :] = dot_b(2 + s % 2)
            b1[0][s] = mk_b1(0, s)
            b1[0][s].start()
            sa2_ref[r0:r0 + seg, :] = dot_a(2 + s % 2)

        a2 = [None] * SEG
        b2 = [None] * SEG
        for s in range(SEG):
            r0 = s * seg
            a1[1][s].wait()
            sa2_ref[r0:r0 + seg, :] = (
                sa2_ref[r0:r0 + seg, :] + ra1_ref[1, r0:r0 + seg, :]
            )
            a2[s] = mk_a2(s)
            a2[s].start()
            b1[1][s].wait()
            sb2_ref[r0:r0 + seg, :] = (
                sb2_ref[r0:r0 + seg, :] + rb1_ref[1, r0:r0 + seg, :]
            )
            b2[s] = mk_b2(s)
            b2[s].start()

        for s in range(SEG):
            r0 = s * seg
            load_seg(my, s, s % 2).wait()
            a1[0][s].wait_send()
            pa_ref[0, r0:r0 + seg, :] = dot_a(s % 2)
            b1[0][s].wait_send()
            pb_ref[0, r0:r0 + seg, :] = dot_b(s % 2)

        out_cps = []
        for s in range(SEG):
            r0 = s * seg
            a1[0][s].wait_recv()
            a2[s].wait()
            sa2_ref[r0:r0 + seg, :] = jnp.maximum(
                pa_ref[0, r0:r0 + seg, :] + ra1_ref[0, r0:r0 + seg, :]
                + ra2_ref[r0:r0 + seg, :], 0.0,
            )
            cp = pltpu.make_async_copy(
                sa2_ref.at[pl.ds(r0, seg), :],
                out_ref.at[pl.ds(r0, seg), pl.ds(0, nh)],
                out_sems.at[0, s],
            )
            cp.start()
            out_cps.append(cp)
            b1[0][s].wait_recv()
            b2[s].wait()
            sb2_ref[r0:r0 + seg, :] = jnp.maximum(
                pb_ref[0, r0:r0 + seg, :] + rb1_ref[0, r0:r0 + seg, :]
                + rb2_ref[r0:r0 + seg, :], 0.0,
            )
            cp = pltpu.make_async_copy(
                sb2_ref.at[pl.ds(r0, seg), :],
                out_ref.at[pl.ds(r0, seg), pl.ds(nh, nh)],
                out_sems.at[1, s],
            )
            cp.start()
            out_cps.append(cp)
        for cp in out_cps:
            cp.wait()

    return pl.pallas_call(
        body,
        out_shape=jax.ShapeDtypeStruct((m_chunk, n), f32),
        in_specs=[
            pl.BlockSpec(memory_space=pltpu.MemorySpace.HBM),
            pl.BlockSpec(memory_space=pltpu.VMEM),
        ],
        out_specs=pl.BlockSpec(memory_space=pltpu.MemorySpace.HBM),
        scratch_shapes=[
            pltpu.VMEM((2, m_chunk, nh), f32),
            pltpu.VMEM((2, m_chunk, nh), f32),
            pltpu.VMEM((2, m_chunk, nh), f32),
            pltpu.VMEM((2, m_chunk, nh), f32),
            pltpu.VMEM((m_chunk, nh), f32),
            pltpu.VMEM((m_chunk, nh), f32),
            pltpu.VMEM((m_chunk, nh), f32),
            pltpu.VMEM((m_chunk, nh), f32),
            pltpu.VMEM((4, seg, k_local), f32),
            pltpu.SemaphoreType.DMA((2, SEG)),
            pltpu.SemaphoreType.DMA((2, SEG)),
            pltpu.SemaphoreType.DMA((2, SEG)),
            pltpu.SemaphoreType.DMA((2, SEG)),
            pltpu.SemaphoreType.DMA((SEG,)),
            pltpu.SemaphoreType.DMA((SEG,)),
            pltpu.SemaphoreType.DMA((SEG,)),
            pltpu.SemaphoreType.DMA((SEG,)),
            pltpu.SemaphoreType.DMA((4,)),
            pltpu.SemaphoreType.DMA((2, SEG)),
        ],
        compiler_params=pltpu.CompilerParams(
            collective_id=0,
            vmem_limit_bytes=128 * 1024 * 1024,
        ),
    )(x, w_mat)
